# speedup vs baseline: 10.1002x; 10.1002x over previous
import functools

import jax
import jax.numpy as jnp
from jax import lax
from jax.experimental import pallas as pl
from jax.experimental.pallas import tpu as pltpu
from jax.experimental.pallas import tpu_sc as plsc

N_NODES = 10000
NPAD = 10240
D = 128
E = 320000
G = 16
NC = 2
NS = 16
NW = NC * NS
CH = 128
KCH = 79
EPAD = NW * KCH * CH
RPT = NPAD // NS
BR = 256
GR = NPAD // BR

_MESH = dict(core_axis_name="c", subcore_axis_name="s", num_cores=NC,
             num_subcores=NS)


@functools.partial(
    pl.kernel,
    out_type=jax.ShapeDtypeStruct((NC, NPAD), jnp.float32),
    mesh=plsc.VectorSubcoreMesh(**_MESH),
    scratch_types=[
        pltpu.VMEM((CH,), jnp.int32),
        pltpu.VMEM((CH,), jnp.float32),
        pltpu.VMEM((RPT,), jnp.float32),
        pltpu.VMEM_SHARED((NPAD,), jnp.float32),
    ],
)
def _sc_degree(dst3, out, dstv, onesv, zv, acc):
    c = lax.axis_index("c")
    s = lax.axis_index("s")
    wid = s * NC + c

    def zbody(i, _):
        zv[pl.ds(pl.multiple_of(i * 16, 16), 16)] = jnp.zeros((16,), jnp.float32)
        return 0

    lax.fori_loop(0, RPT // 16, zbody, 0)
    for q in range(CH // 16):
        onesv[pl.ds(q * 16, 16)] = jnp.ones((16,), jnp.float32)
    pltpu.sync_copy(zv, acc.at[pl.ds(s * RPT, RPT)])
    plsc.subcore_barrier()

    def ebody(j, _):
        pltpu.sync_copy(dst3.at[wid, j], dstv)
        pltpu.sync_copy(onesv, acc.at[dstv], add=True)
        return 0

    lax.fori_loop(0, KCH, ebody, 0)
    plsc.subcore_barrier()
    sl = pl.ds(s * RPT, RPT)
    pltpu.sync_copy(acc.at[sl], out.at[c, sl])


@functools.partial(
    pl.kernel,
    out_type=jax.ShapeDtypeStruct((NC, NPAD, D), jnp.float32),
    mesh=plsc.VectorSubcoreMesh(**_MESH),
    scratch_types=[
        pltpu.VMEM((CH,), jnp.int32),
        pltpu.VMEM((CH,), jnp.int32),
        pltpu.VMEM((CH, D), jnp.float32),
        pltpu.VMEM_SHARED((NPAD, D), jnp.float32),
        pltpu.SemaphoreType.DMA,
    ],
)
def _sc_aggregate(y, src3, dst3, out, srcv, dstv, rows, acc, sem):
    c = lax.axis_index("c")
    s = lax.axis_index("s")
    wid = s * NC + c

    def zbody(i, _):
        for q in range(D // 16):
            rows[i, pl.ds(q * 16, 16)] = jnp.zeros((16,), jnp.float32)
        return 0

    lax.fori_loop(0, CH, zbody, 0)
    for k in range(RPT // CH):
        pltpu.sync_copy(rows, acc.at[pl.ds(s * RPT + k * CH, CH)])
    plsc.subcore_barrier()

    def ebody(j, _):
        pltpu.sync_copy(src3.at[wid, j], srcv)
        pltpu.sync_copy(dst3.at[wid, j], dstv)
        pltpu.async_copy(y.at[srcv], rows, sem).wait()
        pltpu.sync_copy(rows, acc.at[dstv], add=True)
        return 0

    lax.fori_loop(0, KCH, ebody, 0)
    plsc.subcore_barrier()
    for k in range(RPT // CH):
        sl = pl.ds(s * RPT + k * CH, CH)
        pltpu.sync_copy(acc.at[sl], out.at[c, sl])


def _dinv_col(degp_ref):
    deg = degp_ref[:, 0:1] + degp_ref[:, 1:2] + 1.0
    return lax.rsqrt(deg)


def _tc_prep_body(x_ref, degp_ref, w1_ref, y1_ref):
    dinv = _dinv_col(degp_ref)
    xw = x_ref[:, 0:1] * w1_ref[0:1, :] + x_ref[:, 1:2] * w1_ref[1:2, :]
    y1_ref[...] = xw * dinv


_tc_prep = pl.pallas_call(
    _tc_prep_body,
    grid=(GR,),
    in_specs=[
        pl.BlockSpec((BR, 2), lambda i: (i, 0)),
        pl.BlockSpec((BR, NC), lambda i: (i, 0)),
        pl.BlockSpec((2, D), lambda i: (0, 0)),
    ],
    out_specs=pl.BlockSpec((BR, D), lambda i: (i, 0)),
    out_shape=jax.ShapeDtypeStruct((NPAD, D), jnp.float32),
)


def _tc_mid_body(y1_ref, p_ref, degp_ref, b1_ref, w2_ref, y2_ref):
    dinv = _dinv_col(degp_ref)
    h = (p_ref[0] + p_ref[1] + y1_ref[...]) * dinv + b1_ref[...]
    h = jnp.maximum(h, 0.0)
    y2_ref[...] = jnp.dot(h, w2_ref[...],
                          preferred_element_type=jnp.float32) * dinv


_tc_mid = pl.pallas_call(
    _tc_mid_body,
    grid=(GR,),
    in_specs=[
        pl.BlockSpec((BR, D), lambda i: (i, 0)),
        pl.BlockSpec((NC, BR, D), lambda i: (0, i, 0)),
        pl.BlockSpec((BR, NC), lambda i: (i, 0)),
        pl.BlockSpec((1, D), lambda i: (0, 0)),
        pl.BlockSpec((D, D), lambda i: (0, 0)),
    ],
    out_specs=pl.BlockSpec((BR, D), lambda i: (i, 0)),
    out_shape=jax.ShapeDtypeStruct((NPAD, D), jnp.float32),
)


def _tc_final_body(y2_ref, q_ref, degp_ref, b2_ref, batch_ref, out_ref,
                   sums, counts):
    i = pl.program_id(0)

    @pl.when(i == 0)
    def _():
        sums[...] = jnp.zeros((G, D), jnp.float32)
        counts[...] = jnp.zeros((G, D), jnp.float32)

    dinv = _dinv_col(degp_ref)
    ob = (q_ref[0] + q_ref[1] + y2_ref[...]) * dinv + b2_ref[...]
    bblk = batch_ref[0, 0, :]
    oh = (lax.broadcasted_iota(jnp.int32, (G, BR), 0)
          == bblk[None, :]).astype(jnp.float32)
    sums[...] += jnp.dot(oh, ob, preferred_element_type=jnp.float32)
    counts[...] += jnp.dot(oh, jnp.ones((BR, D), jnp.float32),
                           preferred_element_type=jnp.float32)

    @pl.when(i == GR - 1)
    def _():
        out_ref[...] = sums[...] / jnp.maximum(counts[...], 1.0)


_tc_final = pl.pallas_call(
    _tc_final_body,
    grid=(GR,),
    in_specs=[
        pl.BlockSpec((BR, D), lambda i: (i, 0)),
        pl.BlockSpec((NC, BR, D), lambda i: (0, i, 0)),
        pl.BlockSpec((BR, NC), lambda i: (i, 0)),
        pl.BlockSpec((1, D), lambda i: (0, 0)),
        pl.BlockSpec((1, 1, BR), lambda i: (i, 0, 0)),
    ],
    out_specs=pl.BlockSpec((G, D), lambda i: (0, 0)),
    out_shape=jax.ShapeDtypeStruct((G, D), jnp.float32),
    scratch_shapes=[
        pltpu.VMEM((G, D), jnp.float32),
        pltpu.VMEM((G, D), jnp.float32),
    ],
)


@jax.jit
def kernel(invert0, invert1, edge_index, batch, W1, b1, W2, b2):
    f32 = jnp.float32
    x = jnp.stack([invert0, invert1], axis=1).astype(f32)
    x = jnp.pad(x, ((0, NPAD - N_NODES), (0, 0)))
    src = edge_index[0]
    dst = edge_index[1]
    src3 = jnp.pad(src, (0, EPAD - E)).reshape(NW, KCH, CH)
    dst3 = jnp.pad(dst, (0, EPAD - E),
                   constant_values=NPAD - 1).reshape(NW, KCH, CH)
    batch3 = jnp.pad(batch, (0, NPAD - N_NODES),
                     constant_values=G).reshape(GR, 1, BR)
    b1r = b1.reshape(1, D)
    b2r = b2.reshape(1, D)

    degp = _sc_degree(dst3)
    degp_t = degp.T
    y1 = _tc_prep(x, degp_t, W1)
    p = _sc_aggregate(y1, src3, dst3)
    y2 = _tc_mid(y1, p, degp_t, b1r, W2)
    q = _sc_aggregate(y2, src3, dst3)
    return _tc_final(y2, q, degp_t, b2r, batch3)


# speedup vs baseline: 23.8543x; 2.3618x over previous
import functools

import jax
import jax.numpy as jnp
from jax import lax
from jax.experimental import pallas as pl
from jax.experimental.pallas import tpu as pltpu
from jax.experimental.pallas import tpu_sc as plsc

N_NODES = 10000
NPAD = 10240
D = 128
E = 320000
G = 16
NC = 2
NS = 16
NW = NC * NS
CH = 128
KCH = 80
EPAD = NW * KCH * CH
RPT = NPAD // NS
BR = 256
GR = NPAD // BR

_MESH = dict(core_axis_name="c", subcore_axis_name="s", num_cores=NC,
             num_subcores=NS)


@functools.partial(
    pl.kernel,
    out_type=jax.ShapeDtypeStruct((NC, NPAD), jnp.float32),
    mesh=plsc.VectorSubcoreMesh(**_MESH),
    scratch_types=[
        pltpu.VMEM((KCH, CH), jnp.int32),
        pltpu.VMEM((CH,), jnp.float32),
        pltpu.VMEM((RPT,), jnp.float32),
        pltpu.VMEM_SHARED((NPAD,), jnp.float32),
        pltpu.SemaphoreType.DMA,
    ],
)
def _sc_degree(dst3, out, dstall, onesv, zv, acc, dsem):
    c = lax.axis_index("c")
    s = lax.axis_index("s")
    wid = s * NC + c
    pltpu.sync_copy(dst3.at[wid, pl.ds(0, KCH)], dstall)

    def zbody(i, _):
        zv[pl.ds(pl.multiple_of(i * 16, 16), 16)] = jnp.zeros((16,), jnp.float32)
        return 0

    lax.fori_loop(0, RPT // 16, zbody, 0)
    for q in range(CH // 16):
        onesv[pl.ds(q * 16, 16)] = jnp.ones((16,), jnp.float32)
    pltpu.sync_copy(zv, acc.at[pl.ds(s * RPT, RPT)])
    plsc.subcore_barrier()

    def ebody(j, _):
        pltpu.async_copy(onesv, acc.at[dstall.at[j]], dsem, add=True)
        return 0

    lax.fori_loop(0, KCH, ebody, 0)

    def dbody(j, _):
        pltpu.make_async_copy(onesv, acc.at[dstall.at[0]], dsem).wait()
        return 0

    lax.fori_loop(0, KCH, dbody, 0)
    plsc.subcore_barrier()
    sl = pl.ds(s * RPT, RPT)
    pltpu.sync_copy(acc.at[sl], out.at[c, sl])


@functools.partial(
    pl.kernel,
    out_type=jax.ShapeDtypeStruct((NC, NPAD, D), jnp.float32),
    mesh=plsc.VectorSubcoreMesh(**_MESH),
    scratch_types=[
        pltpu.VMEM((KCH + 1, CH), jnp.int32),
        pltpu.VMEM((CH,), jnp.int32),
        pltpu.VMEM((CH,), jnp.int32),
        pltpu.VMEM((CH, D), jnp.float32),
        pltpu.VMEM((CH, D), jnp.float32),
        pltpu.VMEM_SHARED((NPAD, D), jnp.float32),
        pltpu.SemaphoreType.DMA,
        pltpu.SemaphoreType.DMA,
        pltpu.SemaphoreType.DMA,
        pltpu.SemaphoreType.DMA,
        pltpu.SemaphoreType.DMA,
        pltpu.SemaphoreType.DMA,
    ],
)
def _sc_aggregate(y, src3, dst3, out, srcall, dstv0, dstv1, rows0, rows1,
                  acc, gsem0, gsem1, ssem0, ssem1, dsem0, dsem1):
    c = lax.axis_index("c")
    s = lax.axis_index("s")
    wid = s * NC + c
    pltpu.sync_copy(src3.at[wid], srcall)
    pltpu.sync_copy(dst3.at[wid, 0], dstv0)
    pltpu.async_copy(dst3.at[wid, 1], dstv1, dsem1)

    def zbody(i, _):
        for q in range(D // 16):
            rows0[i, pl.ds(q * 16, 16)] = jnp.zeros((16,), jnp.float32)
        return 0

    lax.fori_loop(0, CH, zbody, 0)
    for k in range(RPT // CH):
        pltpu.sync_copy(rows0, acc.at[pl.ds(s * RPT + k * CH, CH)])
    plsc.subcore_barrier()

    def gwait(sem, rows):
        pltpu.make_async_copy(y.at[srcall.at[0]], rows, sem).wait()

    def swait(sem, rows):
        pltpu.make_async_copy(rows, acc.at[dstv0], sem).wait()

    def dwait(sem, dstv):
        pltpu.make_async_copy(dst3.at[wid, 0], dstv, sem).wait()

    pltpu.async_copy(y.at[srcall.at[0]], rows0, gsem0)

    def ebody(i, _):
        j = 2 * i
        gwait(gsem0, rows0)
        pltpu.async_copy(y.at[srcall.at[j + 1]], rows1, gsem1)
        pltpu.async_copy(rows0, acc.at[dstv0], ssem0, add=True)
        dwait(dsem1, dstv1)
        gwait(gsem1, rows1)
        swait(ssem0, rows0)
        pltpu.async_copy(dst3.at[wid, j + 2], dstv0, dsem0)
        pltpu.async_copy(y.at[srcall.at[j + 2]], rows0, gsem0)
        pltpu.async_copy(rows1, acc.at[dstv1], ssem1, add=True)
        swait(ssem1, rows1)
        pltpu.async_copy(dst3.at[wid, j + 3], dstv1, dsem1)
        dwait(dsem0, dstv0)
        return 0

    lax.fori_loop(0, KCH // 2, ebody, 0)
    gwait(gsem0, rows0)
    dwait(dsem1, dstv1)
    plsc.subcore_barrier()
    sl = pl.ds(s * RPT, RPT)
    pltpu.sync_copy(acc.at[sl], out.at[c, sl])


def _dinv_col(degp_ref):
    deg = degp_ref[:, 0:1] + degp_ref[:, 1:2] + 1.0
    return lax.rsqrt(deg)


def _tc_prep_body(x_ref, degp_ref, w1_ref, y1_ref):
    dinv = _dinv_col(degp_ref)
    xw = x_ref[:, 0:1] * w1_ref[0:1, :] + x_ref[:, 1:2] * w1_ref[1:2, :]
    y1_ref[...] = xw * dinv


_tc_prep = pl.pallas_call(
    _tc_prep_body,
    grid=(GR,),
    in_specs=[
        pl.BlockSpec((BR, 2), lambda i: (i, 0)),
        pl.BlockSpec((BR, NC), lambda i: (i, 0)),
        pl.BlockSpec((2, D), lambda i: (0, 0)),
    ],
    out_specs=pl.BlockSpec((BR, D), lambda i: (i, 0)),
    out_shape=jax.ShapeDtypeStruct((NPAD, D), jnp.float32),
)


def _tc_mid_body(y1_ref, p_ref, degp_ref, b1_ref, w2_ref, y2_ref):
    dinv = _dinv_col(degp_ref)
    h = (p_ref[0] + p_ref[1] + y1_ref[...]) * dinv + b1_ref[...]
    h = jnp.maximum(h, 0.0)
    y2_ref[...] = jnp.dot(h, w2_ref[...],
                          preferred_element_type=jnp.float32) * dinv


_tc_mid = pl.pallas_call(
    _tc_mid_body,
    grid=(GR,),
    in_specs=[
        pl.BlockSpec((BR, D), lambda i: (i, 0)),
        pl.BlockSpec((NC, BR, D), lambda i: (0, i, 0)),
        pl.BlockSpec((BR, NC), lambda i: (i, 0)),
        pl.BlockSpec((1, D), lambda i: (0, 0)),
        pl.BlockSpec((D, D), lambda i: (0, 0)),
    ],
    out_specs=pl.BlockSpec((BR, D), lambda i: (i, 0)),
    out_shape=jax.ShapeDtypeStruct((NPAD, D), jnp.float32),
)


def _tc_final_body(y2_ref, q_ref, degp_ref, b2_ref, batch_ref, out_ref,
                   sums, counts):
    i = pl.program_id(0)

    @pl.when(i == 0)
    def _():
        sums[...] = jnp.zeros((G, D), jnp.float32)
        counts[...] = jnp.zeros((G, D), jnp.float32)

    dinv = _dinv_col(degp_ref)
    ob = (q_ref[0] + q_ref[1] + y2_ref[...]) * dinv + b2_ref[...]
    bblk = batch_ref[0, 0, :]
    oh = (lax.broadcasted_iota(jnp.int32, (G, BR), 0)
          == bblk[None, :]).astype(jnp.float32)
    sums[...] += jnp.dot(oh, ob, preferred_element_type=jnp.float32)
    counts[...] += jnp.dot(oh, jnp.ones((BR, D), jnp.float32),
                           preferred_element_type=jnp.float32)

    @pl.when(i == GR - 1)
    def _():
        out_ref[...] = sums[...] / jnp.maximum(counts[...], 1.0)


_tc_final = pl.pallas_call(
    _tc_final_body,
    grid=(GR,),
    in_specs=[
        pl.BlockSpec((BR, D), lambda i: (i, 0)),
        pl.BlockSpec((NC, BR, D), lambda i: (0, i, 0)),
        pl.BlockSpec((BR, NC), lambda i: (i, 0)),
        pl.BlockSpec((1, D), lambda i: (0, 0)),
        pl.BlockSpec((1, 1, BR), lambda i: (i, 0, 0)),
    ],
    out_specs=pl.BlockSpec((G, D), lambda i: (0, 0)),
    out_shape=jax.ShapeDtypeStruct((G, D), jnp.float32),
    scratch_shapes=[
        pltpu.VMEM((G, D), jnp.float32),
        pltpu.VMEM((G, D), jnp.float32),
    ],
)


@jax.jit
def kernel(invert0, invert1, edge_index, batch, W1, b1, W2, b2):
    f32 = jnp.float32
    x = jnp.stack([invert0, invert1], axis=1).astype(f32)
    x = jnp.pad(x, ((0, NPAD - N_NODES), (0, 0)))
    src = edge_index[0]
    dst = edge_index[1]
    pad_src = (jnp.arange(EPAD - E, dtype=jnp.int32) * 97) % N_NODES
    src3 = jnp.concatenate([src, pad_src]).reshape(NW, KCH, CH)
    ext_src = ((jnp.arange(NW * CH, dtype=jnp.int32) * 13) % N_NODES)
    src3 = jnp.concatenate(
        [src3, ext_src.reshape(NW, 1, CH)], axis=1)
    pad_dst = N_NODES + (jnp.arange(EPAD - E, dtype=jnp.int32)
                         % (NPAD - N_NODES))
    dst3 = jnp.concatenate([dst, pad_dst]).reshape(NW, KCH, CH)
    dst3 = jnp.concatenate(
        [dst3, jnp.full((NW, 2, CH), NPAD - 1, jnp.int32)], axis=1)
    batch3 = jnp.pad(batch, (0, NPAD - N_NODES),
                     constant_values=G).reshape(GR, 1, BR)
    b1r = b1.reshape(1, D)
    b2r = b2.reshape(1, D)

    degp = _sc_degree(dst3)
    degp_t = degp.T
    y1 = _tc_prep(x, degp_t, W1)
    p = _sc_aggregate(y1, src3, dst3)
    y2 = _tc_mid(y1, p, degp_t, b1r, W2)
    q = _sc_aggregate(y2, src3, dst3)
    return _tc_final(y2, q, degp_t, b2r, batch3)
